# Initial kernel scaffold; baseline (speedup 1.0000x reference)
import jax
import jax.numpy as jnp
from jax.experimental import pallas as pl


def kernel(user, item, gmf_user_emb, gmf_item_emb, mlp_user_emb, mlp_item_emb, W0, b0, W1, b1, W2, b2, W3, b3, Wout, bout):
    raise NotImplementedError("write your pallas kernel here")


# speedup vs baseline: 1.1886x; 1.1886x over previous
import functools

import jax
import jax.numpy as jnp
from jax import lax
from jax.experimental import pallas as pl
from jax.experimental.pallas import tpu as pltpu
from jax.experimental.pallas import tpu_sc as plsc

B = 16384
H = 16
D = 128
CH = 128


def _make_sc_gather():
    info = plsc.get_sparse_core_info()
    nc, ns = info.num_cores, info.num_subcores
    nw = nc * ns
    bpw = B // nw
    nchunk = bpw // CH
    mesh = plsc.VectorSubcoreMesh(core_axis_name="c", subcore_axis_name="s")

    @functools.partial(
        pl.kernel,
        mesh=mesh,
        compiler_params=pltpu.CompilerParams(use_tc_tiling_on_sc=False),
        out_type=[
            jax.ShapeDtypeStruct((B, H), jnp.float32),
            jax.ShapeDtypeStruct((B, H), jnp.float32),
            jax.ShapeDtypeStruct((B, D), jnp.float32),
            jax.ShapeDtypeStruct((B, D), jnp.float32),
        ],
        scratch_types=[
            pltpu.VMEM((nchunk, CH), jnp.int32),
            pltpu.VMEM((nchunk, CH), jnp.int32),
            pltpu.VMEM((CH, H), jnp.float32),
            pltpu.VMEM((CH, H), jnp.float32),
            pltpu.VMEM((CH, D), jnp.float32),
            pltpu.VMEM((CH, D), jnp.float32),
            pltpu.SemaphoreType.DMA,
        ],
    )
    def gather_k(uidx_hbm, iidx_hbm, gu_hbm, gi_hbm, mu_hbm, mi_hbm,
                 ug_out, ig_out, um_out, im_out,
                 uidx_v, iidx_v, gu_v, gi_v, mu_v, mi_v, sem):
        wid = lax.axis_index("s") * nc + lax.axis_index("c")
        base = wid * bpw
        pltpu.sync_copy(uidx_hbm.at[wid], uidx_v)
        pltpu.sync_copy(iidx_hbm.at[wid], iidx_v)
        for c in range(nchunk):
            row = base + c * CH
            pltpu.async_copy(gu_hbm.at[uidx_v.at[c]], gu_v, sem).wait()
            pltpu.sync_copy(gu_v, ug_out.at[pl.ds(row, CH)])
            pltpu.async_copy(gi_hbm.at[iidx_v.at[c]], gi_v, sem).wait()
            pltpu.sync_copy(gi_v, ig_out.at[pl.ds(row, CH)])
            pltpu.async_copy(mu_hbm.at[uidx_v.at[c]], mu_v, sem).wait()
            pltpu.sync_copy(mu_v, um_out.at[pl.ds(row, CH)])
            pltpu.async_copy(mi_hbm.at[iidx_v.at[c]], mi_v, sem).wait()
            pltpu.sync_copy(mi_v, im_out.at[pl.ds(row, CH)])

    return gather_k, nw, nchunk


def _mlp_body(ug_ref, ig_ref, um_ref, im_ref, w0_ref, b0_ref, w1_ref, b1_ref,
              w2_ref, b2_ref, w3_ref, b3_ref, wout_ref, bout_ref, out_ref):
    w0 = w0_ref[...]
    h = jnp.dot(um_ref[...], w0[:D, :], preferred_element_type=jnp.float32)
    h = h + jnp.dot(im_ref[...], w0[D:, :], preferred_element_type=jnp.float32)
    h = jax.nn.relu(h + b0_ref[...])
    h = jax.nn.relu(jnp.dot(h, w1_ref[...], preferred_element_type=jnp.float32) + b1_ref[...])
    h = jax.nn.relu(jnp.dot(h, w2_ref[...], preferred_element_type=jnp.float32) + b2_ref[...])
    h = jax.nn.relu(jnp.dot(h, w3_ref[...], preferred_element_type=jnp.float32) + b3_ref[...])
    gmf = ug_ref[...] * ig_ref[...]
    wout = wout_ref[...]
    z = jnp.dot(gmf, wout[:H, :], preferred_element_type=jnp.float32)
    z = z + jnp.dot(h, wout[H:, :], preferred_element_type=jnp.float32)
    out_ref[...] = jax.nn.sigmoid(z + bout_ref[...])


def _mlp_call(ug, ig, um, im, w0, b0, w1, b1, w2, b2, w3, b3, wout, bout):
    bm = 1024
    grid = (B // bm,)

    def row_block(nc):
        return pl.BlockSpec((bm, nc), lambda i: (i, 0))

    def full_block(a):
        return pl.BlockSpec(a.shape, lambda i: (0,) * a.ndim)

    return pl.pallas_call(
        _mlp_body,
        grid=grid,
        in_specs=[
            row_block(H), row_block(H), row_block(D), row_block(D),
            full_block(w0), full_block(b0), full_block(w1), full_block(b1),
            full_block(w2), full_block(b2), full_block(w3), full_block(b3),
            full_block(wout), full_block(bout),
        ],
        out_specs=pl.BlockSpec((bm, 1), lambda i: (i, 0)),
        out_shape=jax.ShapeDtypeStruct((B, 1), jnp.float32),
    )(ug, ig, um, im, w0, b0, w1, b1, w2, b2, w3, b3, wout, bout)


def kernel(user, item, gmf_user_emb, gmf_item_emb, mlp_user_emb, mlp_item_emb,
           W0, b0, W1, b1, W2, b2, W3, b3, Wout, bout):
    gather_k, nw, nchunk = _make_sc_gather()
    uidx = user.astype(jnp.int32).reshape(nw, nchunk, CH)
    iidx = item.astype(jnp.int32).reshape(nw, nchunk, CH)
    ug, ig, um, im = gather_k(uidx, iidx, gmf_user_emb, gmf_item_emb,
                              mlp_user_emb, mlp_item_emb)
    out = _mlp_call(ug, ig, um, im,
                    W0, b0.reshape(1, -1), W1, b1.reshape(1, -1),
                    W2, b2.reshape(1, -1), W3, b3.reshape(1, -1),
                    Wout, bout.reshape(1, 1))
    return out[:, 0]


# speedup vs baseline: 1.5472x; 1.3018x over previous
import functools

import jax
import jax.numpy as jnp
from jax import lax
from jax.experimental import pallas as pl
from jax.experimental.pallas import tpu as pltpu
from jax.experimental.pallas import tpu_sc as plsc

B = 16384
H = 16
D = 128
CH = 128
U = 100000
RP_COLS = 1024
RP_GRID = (U + RP_COLS - 1) // RP_COLS
U_PAD = RP_GRID * RP_COLS


def _repack_body(xu_ref, xi_ref, ou_ref, oi_ref):
    for ref, o in ((xu_ref, ou_ref), (xi_ref, oi_ref)):
        x = ref[...]
        xs = jnp.concatenate([x[:, c * 128:(c + 1) * 128] for c in range(8)],
                             axis=0)
        o[...] = xs.T


def _repack_call(gt_u, gt_i):
    out = pl.pallas_call(
        _repack_body,
        grid=(RP_GRID,),
        in_specs=[
            pl.BlockSpec((H, RP_COLS), lambda i: (0, i)),
            pl.BlockSpec((H, RP_COLS), lambda i: (0, i)),
        ],
        out_specs=[
            pl.BlockSpec((128, 128), lambda i: (i, 0)),
            pl.BlockSpec((128, 128), lambda i: (i, 0)),
        ],
        out_shape=[
            jax.ShapeDtypeStruct((U_PAD // 8, 128), jnp.float32),
            jax.ShapeDtypeStruct((U_PAD // 8, 128), jnp.float32),
        ],
    )(gt_u, gt_i)
    return out


def _make_sc_mlp_gather(nc, ns):
    nw = nc * ns
    bpw = B // nw
    nchunk = bpw // CH
    mesh = plsc.VectorSubcoreMesh(core_axis_name="c", subcore_axis_name="s")

    @functools.partial(
        pl.kernel,
        mesh=mesh,
        compiler_params=pltpu.CompilerParams(use_tc_tiling_on_sc=False, needs_layout_passes=False),
        out_type=[
            jax.ShapeDtypeStruct((B, D), jnp.float32),
            jax.ShapeDtypeStruct((B, D), jnp.float32),
        ],
        scratch_types=[
            pltpu.VMEM((nchunk, CH), jnp.int32),
            pltpu.VMEM((nchunk, CH), jnp.int32),
            pltpu.VMEM((CH, D), jnp.float32),
            pltpu.VMEM((CH, D), jnp.float32),
            pltpu.VMEM((CH, D), jnp.float32),
            pltpu.VMEM((CH, D), jnp.float32),
            pltpu.SemaphoreType.DMA,
            pltpu.SemaphoreType.DMA,
            pltpu.SemaphoreType.DMA,
            pltpu.SemaphoreType.DMA,
        ],
    )
    def gather_k(uidx_hbm, iidx_hbm, mu_hbm, mi_hbm,
                 um_out, im_out,
                 uidx_v, iidx_v, u0, u1, i0, i1, su0, su1, si0, si1):
        wid = lax.axis_index("s") * nc + lax.axis_index("c")
        base = wid * bpw
        pltpu.sync_copy(uidx_hbm.at[wid], uidx_v)
        pltpu.sync_copy(iidx_hbm.at[wid], iidx_v)
        ubuf, ibuf = (u0, u1), (i0, i1)
        usem, isem = (su0, su1), (si0, si1)
        cps = {}
        for c in range(2):
            cps[("u", c)] = pltpu.async_copy(
                mu_hbm.at[uidx_v.at[c]], ubuf[c % 2], usem[c % 2])
            cps[("i", c)] = pltpu.async_copy(
                mi_hbm.at[iidx_v.at[c]], ibuf[c % 2], isem[c % 2])
        for c in range(nchunk):
            row = base + c * CH
            cps[("u", c)].wait()
            pltpu.sync_copy(ubuf[c % 2], um_out.at[pl.ds(row, CH)])
            if c + 2 < nchunk:
                cps[("u", c + 2)] = pltpu.async_copy(
                    mu_hbm.at[uidx_v.at[c + 2]], ubuf[c % 2], usem[c % 2])
            cps[("i", c)].wait()
            pltpu.sync_copy(ibuf[c % 2], im_out.at[pl.ds(row, CH)])
            if c + 2 < nchunk:
                cps[("i", c + 2)] = pltpu.async_copy(
                    mi_hbm.at[iidx_v.at[c + 2]], ibuf[c % 2], isem[c % 2])

    return gather_k


def _make_sc_gmf(nc, ns):
    nw = nc * ns
    bpw = B // nw
    nchunk = bpw // CH
    ngrp = bpw // 16
    mesh = plsc.VectorSubcoreMesh(core_axis_name="c", subcore_axis_name="s")

    @functools.partial(
        pl.kernel,
        mesh=mesh,
        compiler_params=pltpu.CompilerParams(use_tc_tiling_on_sc=False, needs_layout_passes=False),
        out_type=jax.ShapeDtypeStruct((B,), jnp.float32),
        scratch_types=[
            pltpu.VMEM((nchunk, CH), jnp.int32),
            pltpu.VMEM((nchunk, CH), jnp.int32),
            pltpu.VMEM((nchunk, CH), jnp.int32),
            pltpu.VMEM((nchunk, CH), jnp.int32),
            pltpu.VMEM((bpw, H), jnp.float32),
            pltpu.VMEM((bpw, H), jnp.float32),
            pltpu.VMEM((H,), jnp.float32),
            pltpu.VMEM((H,), jnp.float32),
            pltpu.VMEM((bpw,), jnp.float32),
            pltpu.SemaphoreType.DMA,
        ],
    )
    def gmf_k(uidx_hbm, iidx_hbm, gu_hbm, gi_hbm, wg_hbm, bo_hbm,
              zg_out,
              uidx_v, iidx_v, tu_v, ti_v, gu_v, gi_v, wg_v, bo_v, zg_v, sem):
        wid = lax.axis_index("s") * nc + lax.axis_index("c")
        base = wid * bpw
        pltpu.sync_copy(uidx_hbm.at[wid], uidx_v)
        pltpu.sync_copy(iidx_hbm.at[wid], iidx_v)
        pltpu.sync_copy(wg_hbm, wg_v)
        pltpu.sync_copy(bo_hbm, bo_v)
        for c in range(nchunk):
            for o in range(CH // 16):
                for src, dst in ((uidx_v, tu_v), (iidx_v, ti_v)):
                    v = src[c, pl.ds(o * 16, 16)]
                    w = ((v >> 10) * 1024 + (v & 127) * 8 + ((v >> 7) & 7))
                    dst[c, pl.ds(o * 16, 16)] = w
        cps = []
        for c in range(nchunk):
            cps.append(pltpu.async_copy(
                gu_hbm.at[tu_v.at[c]], gu_v.at[pl.ds(c * CH, CH)], sem))
            cps.append(pltpu.async_copy(
                gi_hbm.at[ti_v.at[c]], gi_v.at[pl.ds(c * CH, CH)], sem))
        for cp in cps:
            cp.wait()
        boutv = bo_v[...]
        wg_cols = [plsc.load_gather(wg_v, [jnp.full((16,), k, jnp.int32)])
                   for k in range(H)]
        iota16 = lax.iota(jnp.int32, 16)

        def grp(g, _):
            ridx = g * 16 + iota16
            acc = boutv
            for k in range(H):
                cidx = jnp.full((16,), k, jnp.int32)
                ucol = plsc.load_gather(gu_v, [ridx, cidx])
                icol = plsc.load_gather(gi_v, [ridx, cidx])
                acc = acc + ucol * icol * wg_cols[k]
            zg_v[pl.ds(g * 16, 16)] = acc
            return ()

        lax.fori_loop(0, ngrp, grp, (), unroll=False)
        pltpu.sync_copy(zg_v, zg_out.at[pl.ds(base, bpw)])

    return gmf_k


def _mlp_body(um_ref, im_ref, zg_ref, w0_ref, b0_ref, w1_ref, b1_ref,
              w2_ref, b2_ref, w3_ref, b3_ref, wx_ref, out_ref):
    w0 = w0_ref[...]
    h = jnp.dot(um_ref[...], w0[:D, :], preferred_element_type=jnp.float32)
    h = h + jnp.dot(im_ref[...], w0[D:, :], preferred_element_type=jnp.float32)
    h = jax.nn.relu(h + b0_ref[...])
    h = jax.nn.relu(jnp.dot(h, w1_ref[...], preferred_element_type=jnp.float32) + b1_ref[...])
    h = jax.nn.relu(jnp.dot(h, w2_ref[...], preferred_element_type=jnp.float32) + b2_ref[...])
    h = jax.nn.relu(jnp.dot(h, w3_ref[...], preferred_element_type=jnp.float32) + b3_ref[...])
    zm = jnp.sum(h * wx_ref[...], axis=1)
    out_ref[...] = jax.nn.sigmoid(zm + zg_ref[...])


def _mlp_call(um, im, zg, w0, b0, w1, b1, w2, b2, w3, b3, wx):
    bm = 1024
    grid = (B // bm,)

    def full_block(a):
        return pl.BlockSpec(a.shape, lambda i: (0,) * a.ndim)

    return pl.pallas_call(
        _mlp_body,
        grid=grid,
        in_specs=[
            pl.BlockSpec((bm, D), lambda i: (i, 0)),
            pl.BlockSpec((bm, D), lambda i: (i, 0)),
            pl.BlockSpec((bm,), lambda i: (i,)),
            full_block(w0), full_block(b0), full_block(w1), full_block(b1),
            full_block(w2), full_block(b2), full_block(w3), full_block(b3),
            full_block(wx),
        ],
        out_specs=pl.BlockSpec((bm,), lambda i: (i,)),
        out_shape=jax.ShapeDtypeStruct((B,), jnp.float32),
    )(um, im, zg, w0, b0, w1, b1, w2, b2, w3, b3, wx)


def kernel(user, item, gmf_user_emb, gmf_item_emb, mlp_user_emb, mlp_item_emb,
           W0, b0, W1, b1, W2, b2, W3, b3, Wout, bout):
    info = plsc.get_sparse_core_info()
    nc, ns = info.num_cores, info.num_subcores
    nw = nc * ns
    nchunk = B // nw // CH
    uidx = user.astype(jnp.int32).reshape(nw, nchunk, CH)
    iidx = item.astype(jnp.int32).reshape(nw, nchunk, CH)

    um, im = _make_sc_mlp_gather(nc, ns)(uidx, iidx, mlp_user_emb, mlp_item_emb)

    ru, ri = _repack_call(gmf_user_emb.T, gmf_item_emb.T)
    gu = ru.reshape(U_PAD, H)
    gi = ri.reshape(U_PAD, H)

    wg = Wout[:H, 0]
    boutv = jnp.broadcast_to(bout, (H,))
    zg = _make_sc_gmf(nc, ns)(uidx, iidx, gu, gi, wg, boutv)

    wx = Wout[H:, 0].reshape(1, H)
    return _mlp_call(um, im, zg,
                     W0, b0.reshape(1, -1), W1, b1.reshape(1, -1),
                     W2, b2.reshape(1, -1), W3, b3.reshape(1, -1), wx)


# speedup vs baseline: 2.2033x; 1.4240x over previous
import functools

import jax
import jax.numpy as jnp
from jax import lax
from jax.experimental import pallas as pl
from jax.experimental.pallas import tpu as pltpu
from jax.experimental.pallas import tpu_sc as plsc

B = 16384
H = 16
D = 128
CH = 128
U = 100000
RP_COLS = 4096
RP_CW = RP_COLS // 8
RP_GRID = (U + RP_COLS - 1) // RP_COLS
U_PAD = RP_GRID * RP_COLS


def _repack_body(xu_ref, xi_ref, ou_ref, oi_ref):
    for ref, o in ((xu_ref, ou_ref), (xi_ref, oi_ref)):
        x = ref[...]
        xs = jnp.concatenate(
            [x[:, c * RP_CW:(c + 1) * RP_CW] for c in range(8)], axis=0)
        o[...] = xs.T


def _repack_call(gt_u, gt_i):
    out = pl.pallas_call(
        _repack_body,
        grid=(RP_GRID,),
        in_specs=[
            pl.BlockSpec((H, RP_COLS), lambda i: (0, i)),
            pl.BlockSpec((H, RP_COLS), lambda i: (0, i)),
        ],
        out_specs=[
            pl.BlockSpec((RP_CW, 128), lambda i: (i, 0)),
            pl.BlockSpec((RP_CW, 128), lambda i: (i, 0)),
        ],
        out_shape=[
            jax.ShapeDtypeStruct((U_PAD // 8, 128), jnp.float32),
            jax.ShapeDtypeStruct((U_PAD // 8, 128), jnp.float32),
        ],
    )(gt_u, gt_i)
    return out


def _make_sc_mlp_gather(nc, ns):
    nw = nc * ns
    bpw = B // nw
    nchunk = bpw // CH
    mesh = plsc.VectorSubcoreMesh(core_axis_name="c", subcore_axis_name="s")

    @functools.partial(
        pl.kernel,
        mesh=mesh,
        compiler_params=pltpu.CompilerParams(use_tc_tiling_on_sc=False, needs_layout_passes=False),
        out_type=[
            jax.ShapeDtypeStruct((B, D), jnp.float32),
            jax.ShapeDtypeStruct((B, D), jnp.float32),
        ],
        scratch_types=[
            pltpu.VMEM((nchunk, CH), jnp.int32),
            pltpu.VMEM((nchunk, CH), jnp.int32),
            pltpu.VMEM((CH, D), jnp.float32),
            pltpu.VMEM((CH, D), jnp.float32),
            pltpu.VMEM((CH, D), jnp.float32),
            pltpu.VMEM((CH, D), jnp.float32),
            pltpu.SemaphoreType.DMA,
            pltpu.SemaphoreType.DMA,
            pltpu.SemaphoreType.DMA,
            pltpu.SemaphoreType.DMA,
        ],
    )
    def gather_k(uidx_hbm, iidx_hbm, mu_hbm, mi_hbm,
                 um_out, im_out,
                 uidx_v, iidx_v, u0, u1, i0, i1, su0, su1, si0, si1):
        wid = lax.axis_index("s") * nc + lax.axis_index("c")
        base = wid * bpw
        pltpu.sync_copy(uidx_hbm.at[wid], uidx_v)
        pltpu.sync_copy(iidx_hbm.at[wid], iidx_v)
        ubuf, ibuf = (u0, u1), (i0, i1)
        usem, isem = (su0, su1), (si0, si1)
        cps = {}
        for c in range(2):
            cps[("u", c)] = pltpu.async_copy(
                mu_hbm.at[uidx_v.at[c]], ubuf[c % 2], usem[c % 2])
            cps[("i", c)] = pltpu.async_copy(
                mi_hbm.at[iidx_v.at[c]], ibuf[c % 2], isem[c % 2])
        for c in range(nchunk):
            row = base + c * CH
            cps[("u", c)].wait()
            pltpu.sync_copy(ubuf[c % 2], um_out.at[pl.ds(row, CH)])
            if c + 2 < nchunk:
                cps[("u", c + 2)] = pltpu.async_copy(
                    mu_hbm.at[uidx_v.at[c + 2]], ubuf[c % 2], usem[c % 2])
            cps[("i", c)].wait()
            pltpu.sync_copy(ibuf[c % 2], im_out.at[pl.ds(row, CH)])
            if c + 2 < nchunk:
                cps[("i", c + 2)] = pltpu.async_copy(
                    mi_hbm.at[iidx_v.at[c + 2]], ibuf[c % 2], isem[c % 2])

    return gather_k


def _make_sc_gmf(nc, ns):
    nw = nc * ns
    bpw = B // nw
    nchunk = bpw // CH
    ngrp = bpw // 16
    mesh = plsc.VectorSubcoreMesh(core_axis_name="c", subcore_axis_name="s")

    @functools.partial(
        pl.kernel,
        mesh=mesh,
        compiler_params=pltpu.CompilerParams(use_tc_tiling_on_sc=False, needs_layout_passes=False),
        out_type=jax.ShapeDtypeStruct((B,), jnp.float32),
        scratch_types=[
            pltpu.VMEM((nchunk, CH), jnp.int32),
            pltpu.VMEM((nchunk, CH), jnp.int32),
            pltpu.VMEM((nchunk, CH), jnp.int32),
            pltpu.VMEM((nchunk, CH), jnp.int32),
            pltpu.VMEM((bpw, H), jnp.float32),
            pltpu.VMEM((bpw, H), jnp.float32),
            pltpu.VMEM((H,), jnp.float32),
            pltpu.VMEM((H,), jnp.float32),
            pltpu.VMEM((bpw,), jnp.float32),
            pltpu.SemaphoreType.DMA,
        ],
    )
    def gmf_k(uidx_hbm, iidx_hbm, gu_hbm, gi_hbm, wg_hbm, bo_hbm,
              zg_out,
              uidx_v, iidx_v, tu_v, ti_v, gu_v, gi_v, wg_v, bo_v, zg_v, sem):
        wid = lax.axis_index("s") * nc + lax.axis_index("c")
        base = wid * bpw
        pltpu.sync_copy(uidx_hbm.at[wid], uidx_v)
        pltpu.sync_copy(iidx_hbm.at[wid], iidx_v)
        pltpu.sync_copy(wg_hbm, wg_v)
        pltpu.sync_copy(bo_hbm, bo_v)
        for c in range(nchunk):
            for o in range(CH // 16):
                for src, dst in ((uidx_v, tu_v), (iidx_v, ti_v)):
                    v = src[c, pl.ds(o * 16, 16)]
                    w = ((v >> 12) * 4096 + (v & 511) * 8 + ((v >> 9) & 7))
                    dst[c, pl.ds(o * 16, 16)] = w
        cps = []
        for c in range(nchunk):
            cps.append(pltpu.async_copy(
                gu_hbm.at[tu_v.at[c]], gu_v.at[pl.ds(c * CH, CH)], sem))
            cps.append(pltpu.async_copy(
                gi_hbm.at[ti_v.at[c]], gi_v.at[pl.ds(c * CH, CH)], sem))
        for cp in cps:
            cp.wait()
        boutv = bo_v[...]
        wg_cols = [plsc.load_gather(wg_v, [jnp.full((16,), k, jnp.int32)])
                   for k in range(H)]
        iota16 = lax.iota(jnp.int32, 16)

        def grp(g, _):
            ridx = g * 16 + iota16
            acc = boutv
            for k in range(H):
                cidx = jnp.full((16,), k, jnp.int32)
                ucol = plsc.load_gather(gu_v, [ridx, cidx])
                icol = plsc.load_gather(gi_v, [ridx, cidx])
                acc = acc + ucol * icol * wg_cols[k]
            zg_v[pl.ds(g * 16, 16)] = acc
            return ()

        lax.fori_loop(0, ngrp, grp, (), unroll=False)
        pltpu.sync_copy(zg_v, zg_out.at[pl.ds(base, bpw)])

    return gmf_k


def _bf(x):
    return x.astype(jnp.bfloat16)


def _mlp_body(um_ref, im_ref, zg_ref, w0_ref, b0_ref, w1_ref, b1_ref,
              w2_ref, b2_ref, w3_ref, b3_ref, wx_ref, out_ref):
    w0 = w0_ref[...]
    h = jnp.dot(_bf(um_ref[...]), _bf(w0[:D, :]),
                preferred_element_type=jnp.float32)
    h = h + jnp.dot(_bf(im_ref[...]), _bf(w0[D:, :]),
                    preferred_element_type=jnp.float32)
    h = jax.nn.relu(h + b0_ref[...])
    for w_ref, b_ref in ((w1_ref, b1_ref), (w2_ref, b2_ref), (w3_ref, b3_ref)):
        h = jax.nn.relu(jnp.dot(_bf(h), _bf(w_ref[...]),
                                preferred_element_type=jnp.float32) + b_ref[...])
    zm = jnp.sum(h * wx_ref[...], axis=1)
    out_ref[...] = jax.nn.sigmoid(zm + zg_ref[...])


def _mlp_call(um, im, zg, w0, b0, w1, b1, w2, b2, w3, b3, wx):
    bm = 1024
    grid = (B // bm,)

    def full_block(a):
        return pl.BlockSpec(a.shape, lambda i: (0,) * a.ndim)

    return pl.pallas_call(
        _mlp_body,
        grid=grid,
        in_specs=[
            pl.BlockSpec((bm, D), lambda i: (i, 0)),
            pl.BlockSpec((bm, D), lambda i: (i, 0)),
            pl.BlockSpec((bm,), lambda i: (i,)),
            full_block(w0), full_block(b0), full_block(w1), full_block(b1),
            full_block(w2), full_block(b2), full_block(w3), full_block(b3),
            full_block(wx),
        ],
        out_specs=pl.BlockSpec((bm,), lambda i: (i,)),
        out_shape=jax.ShapeDtypeStruct((B,), jnp.float32),
    )(um, im, zg, w0, b0, w1, b1, w2, b2, w3, b3, wx)


def kernel(user, item, gmf_user_emb, gmf_item_emb, mlp_user_emb, mlp_item_emb,
           W0, b0, W1, b1, W2, b2, W3, b3, Wout, bout):
    info = plsc.get_sparse_core_info()
    nc, ns = info.num_cores, info.num_subcores
    nw = nc * ns
    nchunk = B // nw // CH
    uidx = user.astype(jnp.int32).reshape(nw, nchunk, CH)
    iidx = item.astype(jnp.int32).reshape(nw, nchunk, CH)

    ru, ri = _repack_call(gmf_user_emb.T, gmf_item_emb.T)
    gu = ru.reshape(U_PAD, H)
    gi = ri.reshape(U_PAD, H)

    wg = Wout[:H, 0]
    boutv = jnp.broadcast_to(bout, (H,))
    zg = _make_sc_gmf(nc, ns)(uidx, iidx, gu, gi, wg, boutv)

    um, im = _make_sc_mlp_gather(nc, ns)(uidx, iidx, mlp_user_emb, mlp_item_emb)

    wx = Wout[H:, 0].reshape(1, H)
    return _mlp_call(um, im, zg,
                     W0, b0.reshape(1, -1), W1, b1.reshape(1, -1),
                     W2, b2.reshape(1, -1), W3, b3.reshape(1, -1), wx)


# speedup vs baseline: 2.6159x; 1.1873x over previous
import functools

import jax
import jax.numpy as jnp
from jax import lax
from jax.experimental import pallas as pl
from jax.experimental.pallas import tpu as pltpu
from jax.experimental.pallas import tpu_sc as plsc

B = 16384
H = 16
D = 128
CH = 128
U = 100000
RP_COLS = 8192
RP_CW = RP_COLS // 8
RP_GRID = (U + RP_COLS - 1) // RP_COLS
U_PAD = RP_GRID * RP_COLS


def _repack_body(xu_ref, xi_ref, ou_ref, oi_ref):
    for ref, o in ((xu_ref, ou_ref), (xi_ref, oi_ref)):
        x = ref[...]
        xs = jnp.concatenate(
            [x[:, c * RP_CW:(c + 1) * RP_CW] for c in range(8)], axis=0)
        o[...] = xs.T


def _repack_call(gt_u, gt_i):
    out = pl.pallas_call(
        _repack_body,
        grid=(RP_GRID,),
        in_specs=[
            pl.BlockSpec((H, RP_COLS), lambda i: (0, i)),
            pl.BlockSpec((H, RP_COLS), lambda i: (0, i)),
        ],
        out_specs=[
            pl.BlockSpec((RP_CW, 128), lambda i: (i, 0)),
            pl.BlockSpec((RP_CW, 128), lambda i: (i, 0)),
        ],
        out_shape=[
            jax.ShapeDtypeStruct((U_PAD // 8, 128), jnp.float32),
            jax.ShapeDtypeStruct((U_PAD // 8, 128), jnp.float32),
        ],
    )(gt_u, gt_i)
    return out


def _make_sc_mlp_gather(nc, ns):
    nw = nc * ns
    bpw = B // nw
    nchunk = bpw // CH
    mesh = plsc.VectorSubcoreMesh(core_axis_name="c", subcore_axis_name="s")

    @functools.partial(
        pl.kernel,
        mesh=mesh,
        compiler_params=pltpu.CompilerParams(use_tc_tiling_on_sc=False, needs_layout_passes=False),
        cost_estimate=pl.CostEstimate(
            flops=0, bytes_accessed=4 * B * D * 4, transcendentals=0),
        out_type=[
            jax.ShapeDtypeStruct((B, D), jnp.float32),
            jax.ShapeDtypeStruct((B, D), jnp.float32),
        ],
        scratch_types=[
            pltpu.VMEM((nchunk, CH), jnp.int32),
            pltpu.VMEM((nchunk, CH), jnp.int32),
            pltpu.VMEM((CH, D), jnp.float32),
            pltpu.VMEM((CH, D), jnp.float32),
            pltpu.VMEM((CH, D), jnp.float32),
            pltpu.VMEM((CH, D), jnp.float32),
            pltpu.SemaphoreType.DMA,
            pltpu.SemaphoreType.DMA,
            pltpu.SemaphoreType.DMA,
            pltpu.SemaphoreType.DMA,
        ],
    )
    def gather_k(uidx_hbm, iidx_hbm, mu_hbm, mi_hbm,
                 um_out, im_out,
                 uidx_v, iidx_v, u0, u1, i0, i1, su0, su1, si0, si1):
        wid = lax.axis_index("s") * nc + lax.axis_index("c")
        base = wid * bpw
        pltpu.sync_copy(uidx_hbm.at[wid], uidx_v)
        pltpu.sync_copy(iidx_hbm.at[wid], iidx_v)
        ubuf, ibuf = (u0, u1), (i0, i1)
        usem, isem = (su0, su1), (si0, si1)
        cps = {}
        for c in range(2):
            cps[("u", c)] = pltpu.async_copy(
                mu_hbm.at[uidx_v.at[c]], ubuf[c % 2], usem[c % 2])
            cps[("i", c)] = pltpu.async_copy(
                mi_hbm.at[iidx_v.at[c]], ibuf[c % 2], isem[c % 2])
        for c in range(nchunk):
            row = base + c * CH
            cps[("u", c)].wait()
            pltpu.sync_copy(ubuf[c % 2], um_out.at[pl.ds(row, CH)])
            if c + 2 < nchunk:
                cps[("u", c + 2)] = pltpu.async_copy(
                    mu_hbm.at[uidx_v.at[c + 2]], ubuf[c % 2], usem[c % 2])
            cps[("i", c)].wait()
            pltpu.sync_copy(ibuf[c % 2], im_out.at[pl.ds(row, CH)])
            if c + 2 < nchunk:
                cps[("i", c + 2)] = pltpu.async_copy(
                    mi_hbm.at[iidx_v.at[c + 2]], ibuf[c % 2], isem[c % 2])

    return gather_k


def _make_sc_gmf(nc, ns):
    nw = nc * ns
    bpw = B // nw
    nchunk = bpw // CH
    ngrp = bpw // 16
    mesh = plsc.VectorSubcoreMesh(core_axis_name="c", subcore_axis_name="s")

    @functools.partial(
        pl.kernel,
        mesh=mesh,
        compiler_params=pltpu.CompilerParams(use_tc_tiling_on_sc=False, needs_layout_passes=False),
        out_type=jax.ShapeDtypeStruct((B,), jnp.float32),
        scratch_types=[
            pltpu.VMEM((nchunk, CH), jnp.int32),
            pltpu.VMEM((nchunk, CH), jnp.int32),
            pltpu.VMEM((nchunk, CH), jnp.int32),
            pltpu.VMEM((nchunk, CH), jnp.int32),
            pltpu.VMEM((bpw, H), jnp.float32),
            pltpu.VMEM((bpw, H), jnp.float32),
            pltpu.VMEM((H,), jnp.float32),
            pltpu.VMEM((H,), jnp.float32),
            pltpu.VMEM((bpw,), jnp.float32),
            pltpu.SemaphoreType.DMA,
        ],
    )
    def gmf_k(uidx_hbm, iidx_hbm, gu_hbm, gi_hbm, wg_hbm, bo_hbm,
              zg_out,
              uidx_v, iidx_v, tu_v, ti_v, gu_v, gi_v, wg_v, bo_v, zg_v, sem):
        wid = lax.axis_index("s") * nc + lax.axis_index("c")
        base = wid * bpw
        pltpu.sync_copy(uidx_hbm.at[wid], uidx_v)
        pltpu.sync_copy(iidx_hbm.at[wid], iidx_v)
        pltpu.sync_copy(wg_hbm, wg_v)
        pltpu.sync_copy(bo_hbm, bo_v)
        for c in range(nchunk):
            for o in range(CH // 16):
                for src, dst in ((uidx_v, tu_v), (iidx_v, ti_v)):
                    v = src[c, pl.ds(o * 16, 16)]
                    w = ((v >> 13) * 8192 + (v & 1023) * 8 + ((v >> 10) & 7))
                    dst[c, pl.ds(o * 16, 16)] = w
        cps = []
        for c in range(nchunk):
            cps.append(pltpu.async_copy(
                gu_hbm.at[tu_v.at[c]], gu_v.at[pl.ds(c * CH, CH)], sem))
            cps.append(pltpu.async_copy(
                gi_hbm.at[ti_v.at[c]], gi_v.at[pl.ds(c * CH, CH)], sem))
        boutv = bo_v[...]
        wg_cols = [plsc.load_gather(wg_v, [jnp.full((16,), k, jnp.int32)])
                   for k in range(H)]
        iota16 = lax.iota(jnp.int32, 16)
        gpc = CH // 16

        def grp(g, _):
            ridx = g * 16 + iota16
            acc = boutv
            for k in range(H):
                cidx = jnp.full((16,), k, jnp.int32)
                ucol = plsc.load_gather(gu_v, [ridx, cidx])
                icol = plsc.load_gather(gi_v, [ridx, cidx])
                acc = acc + ucol * icol * wg_cols[k]
            zg_v[pl.ds(g * 16, 16)] = acc
            return ()

        for c in range(nchunk):
            cps[2 * c].wait()
            cps[2 * c + 1].wait()
            lax.fori_loop(c * gpc, (c + 1) * gpc, grp, (), unroll=False)
        pltpu.sync_copy(zg_v, zg_out.at[pl.ds(base, bpw)])

    return gmf_k


def _bf(x):
    return x.astype(jnp.bfloat16)


def _mlp_body(um_ref, im_ref, zg_ref, w0_ref, b0_ref, w1_ref, b1_ref,
              w2_ref, b2_ref, w3_ref, b3_ref, wx_ref, out_ref):
    w0 = w0_ref[...]
    h = jnp.dot(_bf(um_ref[...]), w0[:D, :],
                preferred_element_type=jnp.float32)
    h = h + jnp.dot(_bf(im_ref[...]), w0[D:, :],
                    preferred_element_type=jnp.float32)
    h = jax.nn.relu(h + b0_ref[...])
    for w_ref, b_ref in ((w1_ref, b1_ref), (w2_ref, b2_ref), (w3_ref, b3_ref)):
        h = jax.nn.relu(jnp.dot(_bf(h), w_ref[...],
                                preferred_element_type=jnp.float32) + b_ref[...])
    zm = jnp.sum(h * wx_ref[...], axis=1)
    out_ref[...] = jax.nn.sigmoid(zm + zg_ref[...])


def _mlp_call(um, im, zg, w0, b0, w1, b1, w2, b2, w3, b3, wx):
    bm = 2048
    grid = (B // bm,)

    def full_block(a):
        return pl.BlockSpec(a.shape, lambda i: (0,) * a.ndim)

    return pl.pallas_call(
        _mlp_body,
        grid=grid,
        in_specs=[
            pl.BlockSpec((bm, D), lambda i: (i, 0)),
            pl.BlockSpec((bm, D), lambda i: (i, 0)),
            pl.BlockSpec((bm,), lambda i: (i,)),
            full_block(w0), full_block(b0), full_block(w1), full_block(b1),
            full_block(w2), full_block(b2), full_block(w3), full_block(b3),
            full_block(wx),
        ],
        out_specs=pl.BlockSpec((bm,), lambda i: (i,)),
        out_shape=jax.ShapeDtypeStruct((B,), jnp.float32),
    )(um, im, zg, w0, b0, w1, b1, w2, b2, w3, b3, wx)


def kernel(user, item, gmf_user_emb, gmf_item_emb, mlp_user_emb, mlp_item_emb,
           W0, b0, W1, b1, W2, b2, W3, b3, Wout, bout):
    info = plsc.get_sparse_core_info()
    nc, ns = info.num_cores, info.num_subcores
    nw = nc * ns
    nchunk = B // nw // CH
    uidx = user.astype(jnp.int32).reshape(nw, nchunk, CH)
    iidx = item.astype(jnp.int32).reshape(nw, nchunk, CH)

    ru, ri = _repack_call(gmf_user_emb.T, gmf_item_emb.T)
    gu = ru.reshape(U_PAD, H)
    gi = ri.reshape(U_PAD, H)

    wg = Wout[:H, 0]
    boutv = jnp.broadcast_to(bout, (H,))
    zg = _make_sc_gmf(nc, ns)(uidx, iidx, gu, gi, wg, boutv)

    um, im = _make_sc_mlp_gather(nc, ns)(uidx, iidx, mlp_user_emb, mlp_item_emb)

    wx = Wout[H:, 0].reshape(1, H)
    bf = jnp.bfloat16
    return _mlp_call(um, im, zg,
                     W0.astype(bf), b0.reshape(1, -1), W1.astype(bf),
                     b1.reshape(1, -1), W2.astype(bf), b2.reshape(1, -1),
                     W3.astype(bf), b3.reshape(1, -1), wx)
